# baseline (speedup 1.0000x reference)
import functools

import jax
import jax.numpy as jnp
from jax import lax
from jax.experimental import pallas as pl
from jax.experimental.pallas import tpu as pltpu
from jax.experimental.pallas import tpu_sc as plsc

NC = 2
NS = 16
NW = NC * NS
LANES = 16


def _make_bag_kernel(B, L, E, interpret=False):
    assert L % 2 == 0
    Lh = L // 2
    bpw = B // NW
    CB = 8
    nchunks = bpw // CB
    assert bpw % CB == 0
    nseg = 2 * CB

    mesh = plsc.VectorSubcoreMesh(
        core_axis_name="c", subcore_axis_name="s", num_cores=NC, num_subcores=NS
    )

    @functools.partial(
        pl.kernel,
        out_type=jax.ShapeDtypeStruct((B, E), jnp.float32),
        mesh=mesh,
        scratch_types=[
            pltpu.VMEM((nseg, Lh), jnp.int32),
            pltpu.VMEM((nseg, Lh, E), jnp.float32),
            pltpu.VMEM((CB, E), jnp.float32),
            pltpu.SemaphoreType.DMA,
        ],
        compiler_params=pltpu.CompilerParams(use_tc_tiling_on_sc=False),
        interpret=interpret,
    )
    def bag_kernel(x_hbm, table_hbm, bag_hbm, idx_v, rows_v, bag_v, sem):
        wid = lax.axis_index("s") * NC + lax.axis_index("c")
        base = wid * bpw
        scale = jnp.float32(1.0 / L)

        def chunk(ci, carry):
            off = base + ci * CB
            pltpu.sync_copy(x_hbm.at[pl.ds(2 * off, nseg)], idx_v)
            cps = [
                pltpu.async_copy(table_hbm.at[idx_v.at[s]], rows_v.at[s], sem)
                for s in range(nseg)
            ]
            for cp in cps:
                cp.wait()
            for r in range(CB):
                def red(j, acc):
                    return tuple(
                        acc[c]
                        + rows_v[2 * r, j, pl.ds(LANES * c, LANES)]
                        + rows_v[2 * r + 1, j, pl.ds(LANES * c, LANES)]
                        for c in range(E // LANES)
                    )
                acc0 = tuple(
                    jnp.zeros((LANES,), jnp.float32) for _ in range(E // LANES)
                )
                acc = lax.fori_loop(0, Lh, red, acc0, unroll=2)
                for c in range(E // LANES):
                    bag_v[r, pl.ds(LANES * c, LANES)] = acc[c] * scale
            pltpu.sync_copy(bag_v, bag_hbm.at[pl.ds(off, CB)])
            return carry

        lax.fori_loop(0, nchunks, chunk, 0)

    return bag_kernel


def _proj_bcast(bag, Wt, b2, YLEN):
    B, E = bag.shape
    OUT = Wt.shape[1]
    BT = 256

    def body(bag_ref, w_ref, b_ref, out_ref):
        proj = (
            jnp.dot(bag_ref[...], w_ref[...], preferred_element_type=jnp.float32)
            + b_ref[...]
        )
        cat = jnp.concatenate([proj, proj], axis=1)
        out_ref[...] = jnp.broadcast_to(
            cat[:, None, :], (BT, YLEN // 2, 2 * OUT)
        )

    return pl.pallas_call(
        body,
        grid=(B // BT,),
        in_specs=[
            pl.BlockSpec((BT, E), lambda i: (i, 0)),
            pl.BlockSpec((E, OUT), lambda i: (0, 0)),
            pl.BlockSpec((1, OUT), lambda i: (0, 0)),
        ],
        out_specs=pl.BlockSpec((BT, YLEN // 2, 2 * OUT), lambda i: (i, 0, 0)),
        out_shape=jax.ShapeDtypeStruct((B, YLEN // 2, 2 * OUT), jnp.float32),
    )(bag, Wt, b2)


def kernel(x, y_c, table, W, b):
    B, L = x.shape
    YLEN = y_c.shape[1]
    V, E = table.shape
    OUT = W.shape[0]
    x_r = x.astype(jnp.int32).reshape(2 * B, L // 2)
    bag = _make_bag_kernel(B, L, E)(x_r, table)
    out = _proj_bcast(bag, W.T, b.reshape(1, OUT), YLEN)
    return out.reshape(B, YLEN, OUT)


# speedup vs baseline: 1.0726x; 1.0726x over previous
import functools

import jax
import jax.numpy as jnp
from jax import lax
from jax.experimental import pallas as pl
from jax.experimental.pallas import tpu as pltpu
from jax.experimental.pallas import tpu_sc as plsc

NC = 2
NS = 16
NW = NC * NS
LANES = 16


def _make_transpose_kernel(E, V, interpret=False):
    assert E == 64
    TB = 128
    nblk = V // TB
    tail = V - nblk * TB
    kmax = nblk // NW
    rem = nblk - kmax * NW

    mesh = plsc.VectorSubcoreMesh(
        core_axis_name="c", subcore_axis_name="s", num_cores=NC, num_subcores=NS
    )

    @functools.partial(
        pl.kernel,
        out_type=jax.ShapeDtypeStruct((V // 2, 2 * E), jnp.float32),
        mesh=mesh,
        scratch_types=[
            pltpu.VMEM((8, 8, TB), jnp.float32),
            pltpu.VMEM((8, 8, TB), jnp.float32),
            pltpu.VMEM((TB // 2, 2 * E), jnp.float32),
            pltpu.VMEM((TB // 2, 2 * E), jnp.float32),
            pltpu.SemaphoreType.DMA,
            pltpu.SemaphoreType.DMA,
            pltpu.SemaphoreType.DMA,
        ],
        compiler_params=pltpu.CompilerParams(use_tc_tiling_on_sc=True),
        interpret=interpret,
    )
    def tr_kernel(tt_hbm, out_hbm, blk0, blk1, obt0, obt1, isem0, isem1, osem):
        wid = lax.axis_index("s") * NC + lax.axis_index("c")
        blks = [blk0, blk1]
        obts = [obt0, obt1]
        isems = [isem0, isem1]
        iot = lax.iota(jnp.int32, 16)
        c8 = jnp.full((16,), 8, jnp.int32)
        i_lo = lax.rem(iot, c8)
        i_hi = lax.div(iot, c8)
        rfull = [jnp.full((16,), r, jnp.int32) for r in range(8)]
        colv = [
            jnp.full((16,), 8 * r, jnp.int32)
            + i_lo
            + jnp.full((16,), 64, jnp.int32) * i_hi
            for r in range(8)
        ]

        def issue(c, b):
            for r in range(8):
                pltpu.async_copy(
                    tt_hbm.at[pl.ds(8 * r, 8), pl.ds(TB * c, TB)],
                    blks[b].at[r],
                    isems[b],
                )

        def wait_in(c, b):
            for r in range(8):
                pltpu.make_async_copy(
                    tt_hbm.at[pl.ds(8 * r, 8), pl.ds(TB * c, TB)],
                    blks[b].at[r],
                    isems[b],
                ).wait()

        def transpose(b, ncol):
            def col2(j2, carry):
                row16 = lax.broadcast(j2, (16,))
                i2 = lax.broadcast(2 * j2, (16,)) + i_hi
                for r in range(8):
                    g = plsc.load_gather(blks[b], [rfull[r], i_lo, i2])
                    plsc.store_scatter(obts[b], [row16, colv[r]], g)
                return carry
            lax.fori_loop(0, ncol // 2, col2, 0, unroll=2)

        def wout(c, b, nrow):
            pltpu.async_copy(
                obts[b].at[pl.ds(0, nrow)],
                out_hbm.at[pl.ds(64 * c, nrow)],
                osem,
            )

        def wout_wait(c, b, nrow):
            pltpu.make_async_copy(
                obts[b].at[pl.ds(0, nrow)],
                out_hbm.at[pl.ds(64 * c, nrow)],
                osem,
            ).wait()

        def blkid(k):
            return wid + NW * k

        issue(blkid(0), 0)

        def step(k, carry):
            return carry

        nsteps = kmax
        assert nsteps % 2 == 0

        def pair(i2, carry):
            for b in range(2):
                k = i2 * 2 + b
                c = blkid(k)
                wait_in(c, b)
                @pl.when(k + 1 < nsteps)
                def _():
                    issue(blkid(k + 1), 1 - b)
                @pl.when(k >= 2)
                def _():
                    wout_wait(blkid(k - 2), b, 64)
                transpose(b, TB)
                wout(c, b, 64)
            return carry

        lax.fori_loop(0, nsteps // 2, pair, 0)
        wout_wait(blkid(nsteps - 2), 0, 64)
        wout_wait(blkid(nsteps - 1), 1, 64)

        nblk0 = NW * kmax
        for t in range(rem):
            @pl.when(wid == t)
            def _():
                c = nblk0 + t
                issue(c, 0)
                wait_in(c, 0)
                transpose(0, TB)
                wout(c, 0, 64)
                wout_wait(c, 0, 64)
        if tail:
            @pl.when(wid == rem)
            def _():
                c = nblk
                for r in range(8):
                    pltpu.async_copy(
                        tt_hbm.at[pl.ds(8 * r, 8), pl.ds(TB * c, tail)],
                        blk0.at[r, :, pl.ds(0, tail)],
                        isem0,
                    )
                for r in range(8):
                    pltpu.make_async_copy(
                        tt_hbm.at[pl.ds(8 * r, 8), pl.ds(TB * c, tail)],
                        blk0.at[r, :, pl.ds(0, tail)],
                        isem0,
                    ).wait()
                transpose(0, tail)
                wout(c, 0, tail // 2)
                wout_wait(c, 0, tail // 2)

    return tr_kernel


def _linearize_table(table):
    V, E = table.shape
    y = jnp.reshape(table, (V // 2, 2 * E))
    y = lax.optimization_barrier(y)
    return jnp.reshape(y, (V, E))


def _make_bag_kernel(B, L, E, interpret=False):
    assert L % 2 == 0
    Lh = L // 2
    bpw = B // NW
    CB = 8
    nchunks = bpw // CB
    assert bpw % CB == 0
    nseg = 2 * CB

    mesh = plsc.VectorSubcoreMesh(
        core_axis_name="c", subcore_axis_name="s", num_cores=NC, num_subcores=NS
    )

    @functools.partial(
        pl.kernel,
        out_type=jax.ShapeDtypeStruct((B, E), jnp.float32),
        mesh=mesh,
        scratch_types=[
            pltpu.VMEM((nseg, Lh), jnp.int32),
            pltpu.VMEM((nseg, Lh, E), jnp.float32),
            pltpu.VMEM((CB, E), jnp.float32),
            pltpu.SemaphoreType.DMA,
        ],
        compiler_params=pltpu.CompilerParams(use_tc_tiling_on_sc=False),
        interpret=interpret,
    )
    def bag_kernel(x_hbm, table_hbm, bag_hbm, idx_v, rows_v, bag_v, sem):
        wid = lax.axis_index("s") * NC + lax.axis_index("c")
        base = wid * bpw
        scale = jnp.float32(1.0 / L)

        def chunk(ci, carry):
            off = base + ci * CB
            pltpu.sync_copy(x_hbm.at[pl.ds(2 * off, nseg)], idx_v)
            cps = [
                pltpu.async_copy(table_hbm.at[idx_v.at[s]], rows_v.at[s], sem)
                for s in range(nseg)
            ]
            for cp in cps:
                cp.wait()
            for r in range(CB):
                def red(j, acc):
                    return tuple(
                        acc[c]
                        + rows_v[2 * r, j, pl.ds(LANES * c, LANES)]
                        + rows_v[2 * r + 1, j, pl.ds(LANES * c, LANES)]
                        for c in range(E // LANES)
                    )
                acc0 = tuple(
                    jnp.zeros((LANES,), jnp.float32) for _ in range(E // LANES)
                )
                acc = lax.fori_loop(0, Lh, red, acc0, unroll=2)
                for c in range(E // LANES):
                    bag_v[r, pl.ds(LANES * c, LANES)] = acc[c] * scale
            pltpu.sync_copy(bag_v, bag_hbm.at[pl.ds(off, CB)])
            return carry

        lax.fori_loop(0, nchunks, chunk, 0)

    return bag_kernel


def _proj_bcast(bag, W, b2, YLEN):
    B, E = bag.shape
    OUT = W.shape[0]
    BT = 512

    def body(bag_ref, w_ref, b_ref, out_ref):
        pot = (
            jnp.dot(w_ref[...], bag_ref[...].T, preferred_element_type=jnp.float32)
            + b_ref[...]
        )
        out_ref[...] = jnp.broadcast_to(pot[None, :, :], (YLEN, OUT, BT))

    return pl.pallas_call(
        body,
        grid=(B // BT,),
        in_specs=[
            pl.BlockSpec((BT, E), lambda i: (i, 0)),
            pl.BlockSpec((OUT, E), lambda i: (0, 0)),
            pl.BlockSpec((OUT, 1), lambda i: (0, 0)),
        ],
        out_specs=pl.BlockSpec((YLEN, OUT, BT), lambda i: (0, 0, i)),
        out_shape=jax.ShapeDtypeStruct((YLEN, OUT, B), jnp.float32),
    )(bag, W, b2)


def kernel(x, y_c, table, W, b):
    B, L = x.shape
    YLEN = y_c.shape[1]
    V, E = table.shape
    OUT = W.shape[0]
    table_lin = _linearize_table(table)
    x_r = x.astype(jnp.int32).reshape(2 * B, L // 2)
    bag = _make_bag_kernel(B, L, E)(x_r, table_lin)
    out = _proj_bcast(bag, W, b.reshape(OUT, 1), YLEN)
    return jnp.transpose(out, (2, 0, 1))


# speedup vs baseline: 1.0844x; 1.0111x over previous
import functools

import jax
import jax.numpy as jnp
from jax import lax
from jax.experimental import pallas as pl
from jax.experimental.pallas import tpu as pltpu
from jax.experimental.pallas import tpu_sc as plsc

NC = 2
NS = 16
NW = NC * NS
LANES = 16


def _linearize_table(table):
    V, E = table.shape
    BM = 10000

    def body(a_ref, b_ref, o_ref):
        o_ref[...] = jnp.concatenate([a_ref[...], b_ref[...]], axis=1)

    y = pl.pallas_call(
        body,
        grid=(V // 2 // BM,),
        in_specs=[
            pl.BlockSpec((BM, E), lambda i: (i, 0)),
            pl.BlockSpec((BM, E), lambda i: (i + V // 2 // BM, 0)),
        ],
        out_specs=pl.BlockSpec((BM, 2 * E), lambda i: (i, 0)),
        out_shape=jax.ShapeDtypeStruct((V // 2, 2 * E), jnp.float32),
    )(table, table)
    return jnp.reshape(y, (V, E))


def _make_bag_kernel(B, L, E, V, interpret=False):
    assert L % 2 == 0
    Lh = L // 2
    bpw = B // NW
    CB = 8
    nchunks = bpw // CB
    assert bpw % CB == 0
    nseg = 2 * CB

    mesh = plsc.VectorSubcoreMesh(
        core_axis_name="c", subcore_axis_name="s", num_cores=NC, num_subcores=NS
    )

    @functools.partial(
        pl.kernel,
        out_type=jax.ShapeDtypeStruct((B, E), jnp.float32),
        mesh=mesh,
        scratch_types=[
            pltpu.VMEM((nseg, Lh), jnp.int32),
            pltpu.VMEM((nseg, Lh, E), jnp.float32),
            pltpu.VMEM((CB, E), jnp.float32),
            pltpu.SemaphoreType.DMA,
        ],
        compiler_params=pltpu.CompilerParams(use_tc_tiling_on_sc=False),
        interpret=interpret,
    )
    def bag_kernel(x_hbm, table_hbm, bag_hbm, idx_v, rows_v, bag_v, sem):
        wid = lax.axis_index("s") * NC + lax.axis_index("c")
        base = wid * bpw
        scale = jnp.float32(1.0 / L)
        half = jnp.full((LANES,), V // 2, jnp.int32)
        vm1 = jnp.full((LANES,), V - 1, jnp.int32)
        lane = lax.iota(jnp.int32, LANES)
        nfull = Lh // LANES
        tail0 = Lh - LANES
        ntrans = nfull * LANES - tail0

        def permute_idx():
            def perm(t):
                t2 = t + t
                return jnp.where(t < half, t2, t2 - vm1)

            def row(s, carry):
                def one(i, c2):
                    t = idx_v[s, pl.ds(LANES * i, LANES)]
                    idx_v[s, pl.ds(LANES * i, LANES)] = perm(t)
                    return c2
                lax.fori_loop(0, nfull, one, 0, unroll=2)
                if tail0 % LANES:
                    t = idx_v[s, pl.ds(tail0, LANES)]
                    idx_v[s, pl.ds(tail0, LANES)] = jnp.where(
                        lane < ntrans, t, perm(t)
                    )
                return carry

            lax.fori_loop(0, nseg, row, 0)

        def chunk(ci, carry):
            off = base + ci * CB
            pltpu.sync_copy(x_hbm.at[pl.ds(2 * off, nseg)], idx_v)
            permute_idx()
            cps = [
                pltpu.async_copy(table_hbm.at[idx_v.at[s]], rows_v.at[s], sem)
                for s in range(nseg)
            ]
            for cp in cps:
                cp.wait()
            for r in range(CB):
                def red(j, acc):
                    return tuple(
                        acc[c]
                        + rows_v[2 * r, j, pl.ds(LANES * c, LANES)]
                        + rows_v[2 * r + 1, j, pl.ds(LANES * c, LANES)]
                        for c in range(E // LANES)
                    )
                acc0 = tuple(
                    jnp.zeros((LANES,), jnp.float32) for _ in range(E // LANES)
                )
                acc = lax.fori_loop(0, Lh, red, acc0, unroll=2)
                for c in range(E // LANES):
                    bag_v[r, pl.ds(LANES * c, LANES)] = acc[c] * scale
            pltpu.sync_copy(bag_v, bag_hbm.at[pl.ds(off, CB)])
            return carry

        lax.fori_loop(0, nchunks, chunk, 0)

    return bag_kernel


def _proj_bcast(bag, W, b2, YLEN):
    B, E = bag.shape
    OUT = W.shape[0]
    BT = 512

    def body(bag_ref, w_ref, b_ref, out_ref):
        pot = (
            jnp.dot(w_ref[...], bag_ref[...].T, preferred_element_type=jnp.float32)
            + b_ref[...]
        )
        out_ref[...] = jnp.broadcast_to(pot[None, :, :], (YLEN, OUT, BT))

    return pl.pallas_call(
        body,
        grid=(B // BT,),
        in_specs=[
            pl.BlockSpec((BT, E), lambda i: (i, 0)),
            pl.BlockSpec((OUT, E), lambda i: (0, 0)),
            pl.BlockSpec((OUT, 1), lambda i: (0, 0)),
        ],
        out_specs=pl.BlockSpec((YLEN, OUT, BT), lambda i: (0, 0, i)),
        out_shape=jax.ShapeDtypeStruct((YLEN, OUT, B), jnp.float32),
    )(bag, W, b2)


def kernel(x, y_c, table, W, b):
    B, L = x.shape
    YLEN = y_c.shape[1]
    V, E = table.shape
    OUT = W.shape[0]
    table_lin = _linearize_table(table)
    x_r = x.astype(jnp.int32).reshape(2 * B, L // 2)
    bag = _make_bag_kernel(B, L, E, V)(x_r, table_lin)
    out = _proj_bcast(bag, W, b.reshape(OUT, 1), YLEN)
    return jnp.transpose(out, (2, 0, 1))


# speedup vs baseline: 1.6333x; 1.5061x over previous
import functools

import jax
import jax.numpy as jnp
from jax import lax
from jax.experimental import pallas as pl
from jax.experimental.pallas import tpu as pltpu
from jax.experimental.pallas import tpu_sc as plsc

NC = 2
NS = 16
NW = NC * NS
LANES = 16


CH = 4608
CHH = CH // 2
VMAIN = 999936
VTAIL = 64


def _linearize_table(table):
    V, E = table.shape
    nmain = VMAIN // CH
    ng = nmain + 1
    tail = jnp.reshape(
        lax.slice(table, (VMAIN, 0), (V, 0 + E)), (VTAIL // 2, 2 * E)
    )

    def body(tt_ref, tail_ref, o_ref, lo0, lo1, sem0, sem1):
        i = pl.program_id(0)
        los = [lo0, lo1]
        sems = [sem0, sem1]

        def cp(j, b):
            return pltpu.make_async_copy(
                tt_ref.at[:, pl.ds(CH * j, CH)], los[b], sems[b]
            )

        @pl.when(i == 0)
        def _():
            cp(0, 0).start()

        def work(b):
            @pl.when(i + 1 < nmain)
            def _():
                cp(i + 1, 1 - b).start()
            cp(i, b).wait()
            t_all = los[b][...].T
            o_ref[...] = jnp.concatenate(
                [t_all[:CHH], t_all[CHH:]], axis=1
            )

        @pl.when(jnp.logical_and(i % 2 == 0, i < nmain))
        def _():
            work(0)

        @pl.when(jnp.logical_and(i % 2 == 1, i < nmain))
        def _():
            work(1)

        @pl.when(i == nmain)
        def _():
            o_ref[pl.ds(0, VTAIL // 2), :] = tail_ref[...]
            o_ref[pl.ds(VTAIL // 2, CHH - VTAIL // 2), :] = jnp.zeros(
                (CHH - VTAIL // 2, 2 * E), jnp.float32
            )

    y = pl.pallas_call(
        body,
        grid=(ng,),
        in_specs=[
            pl.BlockSpec(memory_space=pl.ANY),
            pl.BlockSpec((VTAIL // 2, 2 * E), lambda i: (0, 0)),
        ],
        out_specs=pl.BlockSpec((CHH, 2 * E), lambda i: (i, 0)),
        out_shape=jax.ShapeDtypeStruct((ng * CHH, 2 * E), jnp.float32),
        scratch_shapes=[
            pltpu.VMEM((E, CH), jnp.float32),
            pltpu.VMEM((E, CH), jnp.float32),
            pltpu.SemaphoreType.DMA,
            pltpu.SemaphoreType.DMA,
        ],
    )(table.T, tail)
    return jnp.reshape(y, (ng * CH, E))


def _make_bag_kernel(B, L, E, V, interpret=False):
    assert L % 2 == 0
    Lh = L // 2
    bpw = B // NW
    CB = 8
    nchunks = bpw // CB
    assert bpw % CB == 0
    nseg = 2 * CB

    mesh = plsc.VectorSubcoreMesh(
        core_axis_name="c", subcore_axis_name="s", num_cores=NC, num_subcores=NS
    )

    @functools.partial(
        pl.kernel,
        out_type=jax.ShapeDtypeStruct((B, E), jnp.float32),
        mesh=mesh,
        scratch_types=[
            pltpu.VMEM((nseg, Lh), jnp.int32),
            pltpu.VMEM((nseg, Lh, E), jnp.float32),
            pltpu.VMEM((CB, E), jnp.float32),
            pltpu.SemaphoreType.DMA,
        ],
        compiler_params=pltpu.CompilerParams(use_tc_tiling_on_sc=False),
        interpret=interpret,
    )
    def bag_kernel(x_hbm, table_hbm, bag_hbm, idx_v, rows_v, bag_v, sem):
        wid = lax.axis_index("s") * NC + lax.axis_index("c")
        base = wid * bpw
        scale = jnp.float32(1.0 / L)
        chh = jnp.full((LANES,), CHH, jnp.int32)
        vmain = jnp.full((LANES,), VMAIN, jnp.int32)
        lane = lax.iota(jnp.int32, LANES)
        nfull = Lh // LANES
        tail0 = Lh - LANES
        ntrans = nfull * LANES - tail0

        def permute_idx():
            def perm(t):
                n = jnp.right_shift(t, 9)
                j = jnp.right_shift(n * 29128, 18)
                base = j * (-CH) + t + t
                c = t - j * CH
                v = jnp.where(c < chh, base, base - (CH - 1))
                return jnp.where(t < vmain, v, t)

            def row(s, carry):
                def one(i, c2):
                    t = idx_v[s, pl.ds(LANES * i, LANES)]
                    idx_v[s, pl.ds(LANES * i, LANES)] = perm(t)
                    return c2
                lax.fori_loop(0, nfull, one, 0, unroll=2)
                if tail0 % LANES:
                    t = idx_v[s, pl.ds(tail0, LANES)]
                    idx_v[s, pl.ds(tail0, LANES)] = jnp.where(
                        lane < ntrans, t, perm(t)
                    )
                return carry

            lax.fori_loop(0, nseg, row, 0)

        def chunk(ci, carry):
            off = base + ci * CB
            pltpu.sync_copy(x_hbm.at[pl.ds(2 * off, nseg)], idx_v)
            permute_idx()
            cps = [
                pltpu.async_copy(table_hbm.at[idx_v.at[s]], rows_v.at[s], sem)
                for s in range(nseg)
            ]
            for cp in cps:
                cp.wait()
            for r in range(CB):
                def red(j, acc):
                    return tuple(
                        acc[c]
                        + rows_v[2 * r, j, pl.ds(LANES * c, LANES)]
                        + rows_v[2 * r + 1, j, pl.ds(LANES * c, LANES)]
                        for c in range(E // LANES)
                    )
                acc0 = tuple(
                    jnp.zeros((LANES,), jnp.float32) for _ in range(E // LANES)
                )
                acc = lax.fori_loop(0, Lh, red, acc0, unroll=2)
                for c in range(E // LANES):
                    bag_v[r, pl.ds(LANES * c, LANES)] = acc[c] * scale
            pltpu.sync_copy(bag_v, bag_hbm.at[pl.ds(off, CB)])
            return carry

        lax.fori_loop(0, nchunks, chunk, 0)

    return bag_kernel


def _proj_bcast(bag, W, b2, YLEN):
    B, E = bag.shape
    OUT = W.shape[0]
    BT = 512

    def body(bag_ref, w_ref, b_ref, out_ref):
        pot = (
            jnp.dot(w_ref[...], bag_ref[...].T, preferred_element_type=jnp.float32)
            + b_ref[...]
        )
        out_ref[...] = jnp.broadcast_to(pot[None, :, :], (YLEN, OUT, BT))

    return pl.pallas_call(
        body,
        grid=(B // BT,),
        in_specs=[
            pl.BlockSpec((BT, E), lambda i: (i, 0)),
            pl.BlockSpec((OUT, E), lambda i: (0, 0)),
            pl.BlockSpec((OUT, 1), lambda i: (0, 0)),
        ],
        out_specs=pl.BlockSpec((YLEN, OUT, BT), lambda i: (0, 0, i)),
        out_shape=jax.ShapeDtypeStruct((YLEN, OUT, B), jnp.float32),
    )(bag, W, b2)


def kernel(x, y_c, table, W, b):
    B, L = x.shape
    YLEN = y_c.shape[1]
    V, E = table.shape
    OUT = W.shape[0]
    table_lin = _linearize_table(table)
    x_r = x.astype(jnp.int32).reshape(2 * B, L // 2)
    bag = _make_bag_kernel(B, L, E, V)(x_r, table_lin)
    out = _proj_bcast(bag, W, b.reshape(OUT, 1), YLEN)
    return jnp.transpose(out, (2, 0, 1))
